# baseline (speedup 1.0000x reference)
import functools

import jax
import jax.numpy as jnp
from jax import lax
from jax.experimental import pallas as pl
from jax.experimental.pallas import tpu as pltpu
from jax.experimental.pallas import tpu_sc as plsc

_NC = 2
_NS = 16
_NW = _NC * _NS
_LANES = 16
_EMBED = 64
_TILE = 8
_CHUNK = 64


def _make_kernel(batch, n_rel2):
    bpw = batch // _NW
    n_chunks = bpw // _CHUNK
    mesh = plsc.VectorSubcoreMesh(
        core_axis_name="c", subcore_axis_name="s",
        num_cores=_NC, num_subcores=_NS)

    @functools.partial(
        pl.kernel,
        mesh=mesh,
        compiler_params=pltpu.CompilerParams(
            needs_layout_passes=False, use_tc_tiling_on_sc=True),
        out_type=jax.ShapeDtypeStruct((batch,), jnp.float32),
        scratch_types=[
            pltpu.VMEM((bpw,), jnp.int32),
            pltpu.VMEM((bpw,), jnp.int32),
            pltpu.VMEM((bpw,), jnp.int32),
            pltpu.VMEM((2, _CHUNK, _EMBED), jnp.float32),
            pltpu.VMEM((n_rel2, 2 * _EMBED), jnp.float32),
            pltpu.VMEM((2, _CHUNK, _EMBED), jnp.float32),
            pltpu.VMEM((bpw,), jnp.float32),
            pltpu.SemaphoreType.DMA,
            pltpu.SemaphoreType.DMA,
        ],
    )
    def trans_score(head_hbm, rel_hbm, tail_hbm, ent_hbm, relw_hbm, out_hbm,
                    hidx, ridx, tidx, hbuf, rtab, tbuf, outv, sem0, sem1):
        wid = lax.axis_index("s") * _NC + lax.axis_index("c")
        rt_cp = pltpu.async_copy(relw_hbm, rtab, sem0)
        pltpu.sync_copy(head_hbm.at[wid], hidx)
        pltpu.sync_copy(rel_hbm.at[wid], ridx)
        pltpu.sync_copy(tail_hbm.at[wid], tidx)
        rt_cp.wait()

        lanes = lax.iota(jnp.int32, _LANES)

        def fire(c, slot, sem):
            for g in range(_CHUNK // _LANES):
                base = c * _CHUNK + g * _LANES
                hv = hidx[pl.ds(base, _LANES)]
                tv = tidx[pl.ds(base, _LANES)]
                for l in range(_LANES):
                    i = g * _LANES + l
                    h = hv[l]
                    pltpu.async_copy(
                        ent_hbm.at[h >> 3, h & (_TILE - 1)],
                        hbuf.at[slot, i], sem)
                    t = tv[l]
                    pltpu.async_copy(
                        ent_hbm.at[t >> 3, t & (_TILE - 1)],
                        tbuf.at[slot, i], sem)

        def drain(slot, sem):
            for i in range(_CHUNK):
                pltpu.make_async_copy(
                    ent_hbm.at[0, 0], hbuf.at[slot, i], sem).wait()
                pltpu.make_async_copy(
                    ent_hbm.at[0, 0], tbuf.at[slot, i], sem).wait()

        fire(0, 0, sem0)

        def chunk_body(c, _):
            slot = c % 2

            @pl.when(c + 1 < n_chunks)
            def _():
                @pl.when(slot == 0)
                def _():
                    fire(c + 1, 1, sem1)
                @pl.when(slot == 1)
                def _():
                    fire(c + 1, 0, sem0)

            @pl.when(slot == 0)
            def _():
                drain(0, sem0)
            @pl.when(slot == 1)
            def _():
                drain(1, sem1)

            for g in range(_CHUNK // _LANES):
                base = c * _CHUNK + g * _LANES
                rv = ridx[pl.ds(base, _LANES)]
                sv = jnp.zeros((_LANES,), jnp.float32)
                for r16 in range(_LANES):
                    i = g * _LANES + r16
                    r = rv[r16]
                    rrow = r >> 1
                    rcol = (r & 1) * _EMBED
                    acc = jnp.zeros((_LANES,), jnp.float32)
                    for j in range(_EMBED // _LANES):
                        sl = pl.ds(j * _LANES, _LANES)
                        rsl = pl.ds(rcol + j * _LANES, _LANES)
                        acc = acc + jnp.abs(hbuf[slot, i, sl]
                                            + rtab[rrow, rsl]
                                            - tbuf[slot, i, sl])
                    sv = jnp.where(lanes == r16, jnp.sum(acc), sv)
                outv[pl.ds(base, _LANES)] = sv
            return 0

        lax.fori_loop(0, n_chunks, chunk_body, 0)
        pltpu.sync_copy(outv, out_hbm.at[pl.ds(wid * bpw, bpw)])

    return trans_score


def kernel(head, relation, tail, entity_weight, relation_weight):
    batch = head.shape[0]
    bpw = batch // _NW
    shape2 = (_NW, bpw)
    n_ent, emb = entity_weight.shape
    n_rel = relation_weight.shape[0]
    ent3d = entity_weight.reshape(n_ent // _TILE, _TILE, emb)
    rel2 = relation_weight.reshape(n_rel // 2, 2 * emb)
    fn = _make_kernel(batch, n_rel // 2)
    return fn(head.reshape(shape2), relation.reshape(shape2),
              tail.reshape(shape2), ent3d, rel2)


# speedup vs baseline: 1.0616x; 1.0616x over previous
import functools

import jax
import jax.numpy as jnp
from jax import lax
from jax.experimental import pallas as pl
from jax.experimental.pallas import tpu as pltpu
from jax.experimental.pallas import tpu_sc as plsc

_NC = 2
_NS = 16
_NW = _NC * _NS
_LANES = 16
_EMBED = 64
_TILE = 8
_CHUNK = 32


def _make_kernel(batch, n_rel2):
    bpw = batch // _NW
    n_chunks = bpw // _CHUNK
    mesh = plsc.VectorSubcoreMesh(
        core_axis_name="c", subcore_axis_name="s",
        num_cores=_NC, num_subcores=_NS)

    @functools.partial(
        pl.kernel,
        mesh=mesh,
        compiler_params=pltpu.CompilerParams(
            needs_layout_passes=False, use_tc_tiling_on_sc=True),
        out_type=jax.ShapeDtypeStruct((batch,), jnp.float32),
        scratch_types=[
            pltpu.VMEM((bpw,), jnp.int32),
            pltpu.VMEM((bpw,), jnp.int32),
            pltpu.VMEM((bpw,), jnp.int32),
            pltpu.VMEM((2, _CHUNK, _EMBED), jnp.float32),
            pltpu.VMEM((n_rel2, 2 * _EMBED), jnp.float32),
            pltpu.VMEM((2, _CHUNK, _EMBED), jnp.float32),
            pltpu.VMEM((bpw,), jnp.float32),
            pltpu.SemaphoreType.DMA,
            pltpu.SemaphoreType.DMA,
        ],
    )
    def trans_score(head_hbm, rel_hbm, tail_hbm, ent_hbm, relw_hbm, out_hbm,
                    hidx, ridx, tidx, hbuf, rtab, tbuf, outv, sem0, sem1):
        wid = lax.axis_index("s") * _NC + lax.axis_index("c")
        rt_cp = pltpu.async_copy(relw_hbm, rtab, sem0)
        pltpu.sync_copy(head_hbm.at[wid], hidx)
        pltpu.sync_copy(rel_hbm.at[wid], ridx)
        pltpu.sync_copy(tail_hbm.at[wid], tidx)
        rt_cp.wait()

        lanes = lax.iota(jnp.int32, _LANES)

        def fire(c, slot, sem):
            for g in range(_CHUNK // _LANES):
                base = c * _CHUNK + g * _LANES
                hv = hidx[pl.ds(base, _LANES)]
                tv = tidx[pl.ds(base, _LANES)]
                for l in range(_LANES):
                    i = g * _LANES + l
                    h = hv[l]
                    pltpu.async_copy(
                        ent_hbm.at[h >> 3, h & (_TILE - 1)],
                        hbuf.at[slot, i], sem)
                    t = tv[l]
                    pltpu.async_copy(
                        ent_hbm.at[t >> 3, t & (_TILE - 1)],
                        tbuf.at[slot, i], sem)

        def drain(slot, sem):
            for i in range(_CHUNK):
                pltpu.make_async_copy(
                    ent_hbm.at[0, 0], hbuf.at[slot, i], sem).wait()
                pltpu.make_async_copy(
                    ent_hbm.at[0, 0], tbuf.at[slot, i], sem).wait()

        fire(0, 0, sem0)

        def chunk_body(c, _):
            slot = c % 2

            @pl.when(c + 1 < n_chunks)
            def _():
                @pl.when(slot == 0)
                def _():
                    fire(c + 1, 1, sem1)
                @pl.when(slot == 1)
                def _():
                    fire(c + 1, 0, sem0)

            @pl.when(slot == 0)
            def _():
                drain(0, sem0)
            @pl.when(slot == 1)
            def _():
                drain(1, sem1)

            for g in range(_CHUNK // _LANES):
                base = c * _CHUNK + g * _LANES
                rv = ridx[pl.ds(base, _LANES)]
                sv = jnp.zeros((_LANES,), jnp.float32)
                for r16 in range(_LANES):
                    i = g * _LANES + r16
                    r = rv[r16]
                    rrow = r >> 1
                    rcol = (r & 1) * _EMBED
                    acc = jnp.zeros((_LANES,), jnp.float32)
                    for j in range(_EMBED // _LANES):
                        sl = pl.ds(j * _LANES, _LANES)
                        rsl = pl.ds(rcol + j * _LANES, _LANES)
                        acc = acc + jnp.abs(hbuf[slot, i, sl]
                                            + rtab[rrow, rsl]
                                            - tbuf[slot, i, sl])
                    sv = jnp.where(lanes == r16, jnp.sum(acc), sv)
                outv[pl.ds(base, _LANES)] = sv
            return 0

        lax.fori_loop(0, n_chunks, chunk_body, 0)
        pltpu.sync_copy(outv, out_hbm.at[pl.ds(wid * bpw, bpw)])

    return trans_score


def kernel(head, relation, tail, entity_weight, relation_weight):
    batch = head.shape[0]
    bpw = batch // _NW
    shape2 = (_NW, bpw)
    n_ent, emb = entity_weight.shape
    n_rel = relation_weight.shape[0]
    ent3d = entity_weight.reshape(n_ent // _TILE, _TILE, emb)
    rel2 = relation_weight.reshape(n_rel // 2, 2 * emb)
    fn = _make_kernel(batch, n_rel // 2)
    return fn(head.reshape(shape2), relation.reshape(shape2),
              tail.reshape(shape2), ent3d, rel2)
